# baseline (speedup 1.0000x reference)
import jax
import jax.numpy as jnp
from jax.experimental import pallas as pl
from jax.experimental.pallas import tpu as pltpu

B = 8192
L = 4096
ROWS = 256


def _mask_body(pos_ref, img_ref, out_ref):
    col = jax.lax.broadcasted_iota(jnp.int32, (ROWS, L), 1)
    pos = pos_ref[:, 0:1]
    out_ref[...] = jnp.where(col < pos, img_ref[...], 0.0)


def kernel(images, position):
    pos2d = position.reshape(B, 1)
    return pl.pallas_call(
        _mask_body,
        grid=(B // ROWS,),
        in_specs=[
            pl.BlockSpec((ROWS, 1), lambda i: (i, 0)),
            pl.BlockSpec((ROWS, L), lambda i: (i, 0)),
        ],
        out_specs=pl.BlockSpec((ROWS, L), lambda i: (i, 0)),
        out_shape=jax.ShapeDtypeStruct((B, L), jnp.float32),
    )(pos2d, images)
